# baseline (speedup 1.0000x reference)
import functools

import jax
import jax.numpy as jnp
from jax import lax
from jax.experimental import pallas as pl
from jax.experimental.pallas import tpu as pltpu
from jax.experimental.pallas import tpu_sc as plsc

N1 = 10000
N2 = 2500
E1 = 320000
E2 = 80000
K = 6
C1 = 128
H = 1024
NUM_CLASSES = 14

N2P = 2560
E2P = 81920
N1P = 10240
CH = 64
NT = 16


def _sc_pool_conv2(h1s, dsrow2d, src2d, dst2d, dinv2exp, zrows):
    mesh = plsc.VectorSubcoreMesh(core_axis_name="c", subcore_axis_name="s")

    @functools.partial(
        pl.kernel,
        mesh=mesh,
        out_type=(jax.ShapeDtypeStruct((K, 2, N2P, CH), jnp.float32),
                  jax.ShapeDtypeStruct((2, N2P, CH), jnp.float32)),
        compiler_params=pltpu.CompilerParams(use_tc_tiling_on_sc=False),
        scratch_types=[
            pltpu.VMEM_SHARED((N2P, CH), jnp.float32),
            pltpu.VMEM_SHARED((N2P, CH), jnp.float32),
            pltpu.VMEM((40, 128), jnp.int32),
            pltpu.VMEM((40, 128), jnp.int32),
            pltpu.VMEM((5, 128), jnp.int32),
            pltpu.VMEM((160, CH), jnp.float32),
            pltpu.VMEM((128, CH), jnp.float32),
            pltpu.VMEM((128, CH), jnp.float32),
            pltpu.VMEM((160, CH), jnp.float32),
            pltpu.VMEM((160, CH), jnp.float32),
            pltpu.VMEM((160, CH), jnp.float32),
            pltpu.SemaphoreType.DMA,
            pltpu.SemaphoreType.DMA,
            pltpu.SemaphoreType.DMA,
            pltpu.SemaphoreType.DMA,
        ],
    )
    def body(h1s_r, dsrow_r, src_r, dst_r, dexp_r, z_r, t2_r, u_r,
             sh_u, sh_acc, v_src, v_dst, v_dsrow, v_dexp, v_rows, v_rows2,
             v_w, v_txA, v_txB, sg0, sg1, ss0, ss1):
        c = lax.axis_index("c")
        s = lax.axis_index("s")
        co = c * CH
        r0 = s * 160

        pltpu.sync_copy(src_r.at[s], v_src)
        pltpu.sync_copy(dst_r.at[s], v_dst)
        pltpu.sync_copy(dsrow_r.at[s], v_dsrow)
        pltpu.sync_copy(dexp_r.at[pl.ds(r0, 160), :], v_dexp)
        pltpu.sync_copy(z_r, sh_acc.at[pl.ds(r0, 160), :])
        plsc.subcore_barrier()

        for i in range(5):
            pltpu.sync_copy(
                h1s_r.at[c, pl.ds(s * 640 + i * 128, 128), :], v_rows)
            pltpu.sync_copy(v_rows, sh_acc.at[v_dsrow.at[i]], add=True)
        plsc.subcore_barrier()

        def rowpass(k, w_ref, txA, txB, first):
            scale = -1.0 if k == 1 else -2.0

            def rbody(r, _):
                for m in range(CH // 16):
                    sl = pl.ds(m * 16, 16)
                    w = w_ref[r, sl]
                    dv = v_dexp[r, sl]
                    tx = scale * dv * w
                    if not first and k > 1:
                        tx = tx - txA[r, sl]
                    txA[r, sl] = tx
                    w_ref[r, sl] = dv * tx
                return 0
            lax.fori_loop(0, 160, rbody, 0)
            pltpu.sync_copy(txA, t2_r.at[k, c, pl.ds(r0, 160), :])
            pltpu.sync_copy(w_ref, u_r.at[c, pl.ds(r0, 160), :])

        pltpu.sync_copy(sh_acc.at[pl.ds(r0, 160), :], v_txB)
        pltpu.sync_copy(v_txB, t2_r.at[0, c, pl.ds(r0, 160), :])

        def r0body(r, _):
            for m in range(CH // 16):
                sl = pl.ds(m * 16, 16)
                v_w[r, sl] = v_dexp[r, sl] * v_txB[r, sl]
            return 0
        lax.fori_loop(0, 160, r0body, 0)
        pltpu.sync_copy(v_w, u_r.at[c, pl.ds(r0, 160), :])
        plsc.subcore_barrier()

        txA, txB = v_txA, v_txB
        for k in range(1, K):
            pltpu.sync_copy(z_r, sh_acc.at[pl.ds(r0, 160), :])
            plsc.subcore_barrier()

            pltpu.async_copy(u_r.at[c].at[v_src.at[0]], v_rows, sg0)
            pltpu.async_copy(u_r.at[c].at[v_src.at[1]], v_rows2, sg1)

            def spmm_pair(i, _):
                a = 2 * i
                pltpu.make_async_copy(u_r.at[c].at[v_src.at[a]], v_rows, sg0).wait()
                pltpu.async_copy(v_rows, sh_acc.at[v_dst.at[a]], ss0, add=True)
                pltpu.make_async_copy(
                    u_r.at[c].at[v_src.at[a + 1]], v_rows2, sg1).wait()
                pltpu.async_copy(
                    v_rows2, sh_acc.at[v_dst.at[a + 1]], ss1, add=True)

                @pl.when(i < 19)
                def _nxt():
                    pltpu.make_async_copy(
                        v_rows, sh_acc.at[v_dst.at[a]], ss0).wait()
                    pltpu.async_copy(u_r.at[c].at[v_src.at[a + 2]], v_rows, sg0)
                    pltpu.make_async_copy(
                        v_rows2, sh_acc.at[v_dst.at[a + 1]], ss1).wait()
                    pltpu.async_copy(u_r.at[c].at[v_src.at[a + 3]], v_rows2, sg1)

                @pl.when(i == 19)
                def _fin():
                    pltpu.make_async_copy(
                        v_rows, sh_acc.at[v_dst.at[a]], ss0).wait()
                    pltpu.make_async_copy(
                        v_rows2, sh_acc.at[v_dst.at[a + 1]], ss1).wait()
                return 0
            lax.fori_loop(0, 20, spmm_pair, 0)
            plsc.subcore_barrier()

            pltpu.sync_copy(sh_acc.at[pl.ds(r0, 160), :], v_w)
            rowpass(k, v_w, txA, txB, first=(k == 1))
            txA, txB = txB, txA
            plsc.subcore_barrier()

    return body(h1s, dsrow2d, src2d, dst2d, dinv2exp, zrows)


def _tc_head(T2, W2, b2, Wh, bh, Wlp, blp):
    def body(t2_ref, w2_ref, b2_ref, wh_ref, bh_ref, wl_ref, bl_ref, o_ref):
        acc = jnp.zeros((N2P, 128), jnp.float32)
        for k in range(K):
            w2k = w2_ref[k]
            for c in range(2):
                acc = acc + jax.lax.dot_general(
                    t2_ref[k, c], w2k[c * CH:(c + 1) * CH, :],
                    (((1,), (0,)), ((), ())),
                    preferred_element_type=jnp.float32)
        out2 = acc + b2_ref[...]
        Hm = jax.lax.dot_general(out2, wh_ref[...], (((1,), (0,)), ((), ())),
                                 preferred_element_type=jnp.float32)
        Hm = Hm + bh_ref[...]
        rows = jax.lax.broadcasted_iota(jnp.int32, (N2P, H), 0)
        Hm = jnp.where(rows < N2, Hm, -3.0e38)
        m = jnp.max(Hm, axis=0, keepdims=True)
        Z = jax.lax.dot_general(m, wl_ref[...], (((1,), (0,)), ((), ())),
                                preferred_element_type=jnp.float32)
        o_ref[...] = Z + bl_ref[...]

    return pl.pallas_call(
        body,
        out_shape=jax.ShapeDtypeStruct((1, 128), jnp.float32),
    )(T2, W2, b2.reshape(1, 128), Wh, bh.reshape(1, H), Wlp, blp)


E1P = 327680


def _rsqrt16(x):
    i = plsc.bitcast(x, jnp.int32)
    i = jnp.int32(0x5F3759DF) - jax.lax.shift_right_logical(i, 1)
    y = plsc.bitcast(i, jnp.float32)
    for _ in range(4):
        y = y * (1.5 - 0.5 * x * y * y)
    return jnp.where(x > 0.0, y, 0.0)


def _sc_conv1_prep(xT, src1, dst1, src2, z80, id80, id20):
    mesh = plsc.VectorSubcoreMesh(core_axis_name="c", subcore_axis_name="s")

    @functools.partial(
        pl.kernel,
        mesh=mesh,
        out_type=(jax.ShapeDtypeStruct((K, 3, 80, 128), jnp.float32),
                  jax.ShapeDtypeStruct((N2P, CH), jnp.float32)),
        compiler_params=pltpu.CompilerParams(use_tc_tiling_on_sc=False,
                                             needs_layout_passes=False),
        scratch_types=(
            [pltpu.VMEM_SHARED((80, 128), jnp.float32)] * 6 +
            [pltpu.VMEM((80, 128), jnp.float32)] * 6 +
            [pltpu.VMEM((160, 128), jnp.int32)] * 2 +
            [pltpu.VMEM((1, 80), jnp.int32),
             pltpu.VMEM((1, 20), jnp.int32),
             pltpu.VMEM((5, 128), jnp.float32),
             pltpu.VMEM((5, 128), jnp.float32),
             pltpu.VMEM((5, 128), jnp.float32)] +
            [pltpu.VMEM((5, 128), jnp.float32)] * 6 +
            [pltpu.VMEM((128, CH), jnp.float32)]
        ),
    )
    def body(xT_r, src_r, dst_r, src2_r, z_r, id80_r, id20_r, t1_r, dexp_r,
             shu0, shu1, shu2, sha0, sha1, sha2,
             vx0, vx1, vx2, va0, va1, va2,
             v_src, v_dst, v_id80, v_id20, v_w, v_dinv, v_u,
             txA0, txA1, txA2, txB0, txB1, txB2, v_dexp):
        cidx = lax.axis_index("c")
        s = lax.axis_index("s")
        shu = (shu0, shu1, shu2)
        sha = (sha0, sha1, sha2)
        vx = (vx0, vx1, vx2)
        va = (va0, va1, va2)
        sl5 = pl.ds(5 * s, 5)

        def groups(i, fn):
            for j in range(8):
                g = pl.ds(16 * j, 16)
                s16 = v_src[i, g]
                d16 = v_dst[i, g]
                shi = jax.lax.shift_right_logical(s16, 7)
                slo = jax.lax.bitwise_and(s16, 127)
                dhi = jax.lax.shift_right_logical(d16, 7)
                dlo = jax.lax.bitwise_and(d16, 127)
                fn(shi, slo, dhi, dlo)

        ones16 = jnp.full((16,), 1.0, jnp.float32)

        def conv1_program(chans):
            nch = len(chans)
            pltpu.sync_copy(src_r.at[s], v_src)
            pltpu.sync_copy(dst_r.at[s], v_dst)
            pltpu.sync_copy(id80_r, v_id80)
            pltpu.sync_copy(z_r, va0)

            def histbody(i, _):
                groups(i, lambda shi, slo, dhi, dlo:
                       plsc.addupdate_scatter(va0, [shi, slo], ones16))
                return 0
            lax.fori_loop(0, 160, histbody, 0)
            pltpu.sync_copy(z_r.at[pl.ds(0, 5)], sha0.at[sl5])
            plsc.subcore_barrier()
            pltpu.sync_copy(va0, sha0.at[v_id80.at[0]], add=True)
            plsc.subcore_barrier()
            pltpu.sync_copy(sha0.at[sl5], v_w)

            def newton(r, _):
                for m in range(8):
                    g = pl.ds(16 * m, 16)
                    v_dinv[r, g] = _rsqrt16(v_w[r, g])
                return 0
            lax.fori_loop(0, 5, newton, 0)
            txA = [txA0, txA1, txA2][:nch]
            txB = [txB0, txB1, txB2][:nch]
            for j, ch in enumerate(chans):
                pltpu.sync_copy(xT_r.at[ch, sl5], txB[j])
                pltpu.sync_copy(txB[j], t1_r.at[0, ch, sl5])

                def u0body(r, _, _b=txB[j]):
                    for m in range(8):
                        g = pl.ds(16 * m, 16)
                        v_u[r, g] = v_dinv[r, g] * _b[r, g]
                    return 0
                lax.fori_loop(0, 5, u0body, 0)
                pltpu.sync_copy(v_u, shu[j].at[sl5])
            plsc.subcore_barrier()

            for k in range(1, K):
                for j in range(nch):
                    pltpu.sync_copy(z_r, va[j])
                    pltpu.sync_copy(z_r.at[pl.ds(0, 5)], sha[j].at[sl5])
                    pltpu.sync_copy(shu[j], vx[j])
                plsc.subcore_barrier()

                def edgebody(i, _):
                    def work(shi, slo, dhi, dlo):
                        for j in range(nch):
                            g = plsc.load_gather(vx[j], [shi, slo])
                            plsc.addupdate_scatter(va[j], [dhi, dlo], g)
                    groups(i, work)
                    return 0
                lax.fori_loop(0, 160, edgebody, 0)
                for j in range(nch):
                    pltpu.sync_copy(va[j], sha[j].at[v_id80.at[0]], add=True)
                plsc.subcore_barrier()

                scale = -1.0 if k == 1 else -2.0
                for j, ch in enumerate(chans):
                    pltpu.sync_copy(sha[j].at[sl5], v_w)

                    def rowbody(r, _, _a=txA[j], _k=k):
                        for m in range(8):
                            g = pl.ds(16 * m, 16)
                            tx = scale * v_dinv[r, g] * v_w[r, g]
                            if _k > 1:
                                tx = tx - _a[r, g]
                            _a[r, g] = tx
                            v_u[r, g] = v_dinv[r, g] * tx
                        return 0
                    lax.fori_loop(0, 5, rowbody, 0)
                    pltpu.sync_copy(txA[j], t1_r.at[k, ch, sl5])
                    pltpu.sync_copy(v_u, shu[j].at[sl5])
                txA, txB = txB, txA
                plsc.subcore_barrier()

        @pl.when(cidx == 0)
        def _core0():
            conv1_program([0, 1])

        @pl.when(cidx == 1)
        def _core1():
            pltpu.sync_copy(src2_r.at[s], v_src.at[pl.ds(0, 40)])
            pltpu.sync_copy(id20_r, v_id20)
            pltpu.sync_copy(z_r, va0)

            def histbody(i, _):
                groups(i, lambda shi, slo, dhi, dlo:
                       plsc.addupdate_scatter(va0, [shi, slo], ones16))
                return 0
            lax.fori_loop(0, 40, histbody, 0)

            @pl.when(s < 10)
            def _z():
                pltpu.sync_copy(z_r.at[pl.ds(0, 2)], sha0.at[pl.ds(2 * s, 2)])
            plsc.subcore_barrier()
            pltpu.sync_copy(va0.at[pl.ds(0, 20)], sha0.at[v_id20.at[0]], add=True)
            plsc.subcore_barrier()

            @pl.when(s < 10)
            def _dexp():
                pltpu.sync_copy(sha0.at[pl.ds(2 * s, 2)], v_w.at[pl.ds(0, 2)])
                for r in range(2):
                    for m in range(8):
                        g = pl.ds(16 * m, 16)
                        v_dinv[r, g] = _rsqrt16(v_w[r, g])
                for half in range(2):
                    def expand(j, _):
                        sp = plsc.load_gather(
                            v_dinv, [jnp.full((16,), half, jnp.int32),
                                     jnp.full((16,), j, jnp.int32)])
                        for m in range(CH // 16):
                            v_dexp[j, pl.ds(16 * m, 16)] = sp
                        return 0
                    lax.fori_loop(0, 128, expand, 0)
                    pltpu.sync_copy(
                        v_dexp, dexp_r.at[pl.ds(256 * s + 128 * half, 128), :])
            conv1_program([2])

    return body(xT, src1, dst1, src2, z80, id80, id20)


def _tc_mid(T1r18, W1s, b1, dsvp):
    def bodyfn(x_ref, w_ref, b_ref, d_ref, o_ref):
        h = jax.lax.dot_general(x_ref[...], w_ref[...],
                                (((0,), (0,)), ((), ())),
                                preferred_element_type=jnp.float32)
        o_ref[...] = jnp.maximum(h + b_ref[...], 0.0) * d_ref[...]

    return pl.pallas_call(
        bodyfn,
        out_shape=jax.ShapeDtypeStruct((N1P, 128), jnp.float32),
    )(T1r18, W1s, b1.reshape(1, 128), dsvp)


def kernel(pos, edge_index0, edge_index1, ds_row, ds_col, ds_values,
           W1, b1, W2, b2, Wh, bh, Wl, bl):
    i32 = jnp.int32

    xT = jnp.zeros((3, N1P), jnp.float32).at[:, :N1].set(
        pos.T).reshape(3, 80, 128)
    src1 = jnp.full((E1P,), N1, i32).at[:E1].set(
        edge_index0[0].astype(i32)).reshape(NT, 160, 128)
    dst1 = jnp.full((E1P,), N1, i32).at[:E1].set(
        edge_index0[1].astype(i32)).reshape(NT, 160, 128)
    dsrow2d = jnp.full((N1P,), N2, i32).at[:N1].set(
        ds_row.astype(i32)).reshape(NT, 5, 128)
    src2 = jnp.full((E2P,), N2, i32).at[:E2].set(
        edge_index1[0].astype(i32)).reshape(NT, 40, 128)
    dst2 = jnp.full((E2P,), N2, i32).at[:E2].set(
        edge_index1[1].astype(i32)).reshape(NT, 40, 128)
    z80 = jnp.zeros((80, 128), jnp.float32)
    id80 = jnp.arange(80, dtype=i32).reshape(1, 80)
    id20 = jnp.arange(20, dtype=i32).reshape(1, 20)
    zrows = jnp.zeros((160, CH), jnp.float32)
    dsvp = jnp.zeros((N1P, 1), jnp.float32).at[:N1, 0].set(ds_values)

    T1, dinv2exp = _sc_conv1_prep(xT, src1, dst1, src2, z80, id80, id20)

    W1s = W1.reshape(K * 3, 128)
    h1 = _tc_mid(T1.reshape(K * 3, N1P), W1s, b1, dsvp)
    h1s = jnp.stack([h1[:, :CH], h1[:, CH:]])

    T2, _ = _sc_pool_conv2(h1s, dsrow2d, src2, dst2, dinv2exp, zrows)

    Wlp = jnp.zeros((H, 128), jnp.float32).at[:, :NUM_CLASSES].set(Wl)
    blp = jnp.zeros((1, 128), jnp.float32).at[0, :NUM_CLASSES].set(bl)
    Zp = _tc_head(T2, W2, b2, Wh, bh, Wlp, blp)
    return Zp[0, :NUM_CLASSES]


# speedup vs baseline: 1.4683x; 1.4683x over previous
import functools

import jax
import jax.numpy as jnp
from jax import lax
from jax.experimental import pallas as pl
from jax.experimental.pallas import tpu as pltpu
from jax.experimental.pallas import tpu_sc as plsc

N1 = 10000
N2 = 2500
E1 = 320000
E2 = 80000
K = 6
C1 = 128
H = 1024
NUM_CLASSES = 14

N2P = 2560
E2P = 81920
N1P = 10240
CH = 64
NT = 16


def _sc_pool_conv2(h1s, dsrow2d, src2d, dst2d, dinv2exp, zrows):
    mesh = plsc.VectorSubcoreMesh(core_axis_name="c", subcore_axis_name="s")

    @functools.partial(
        pl.kernel,
        mesh=mesh,
        out_type=jax.ShapeDtypeStruct((K, 2, N2P, CH), jnp.float32),
        compiler_params=pltpu.CompilerParams(use_tc_tiling_on_sc=False),
        scratch_types=[
            pltpu.VMEM_SHARED((N2P, CH), jnp.float32),
            pltpu.VMEM_SHARED((N2P, CH), jnp.float32),
            pltpu.VMEM((40, 128), jnp.int32),
            pltpu.VMEM((40, 128), jnp.int32),
            pltpu.VMEM((5, 128), jnp.int32),
            pltpu.VMEM((160, CH), jnp.float32),
            pltpu.VMEM((128, CH), jnp.float32),
            pltpu.VMEM((128, CH), jnp.float32),
            pltpu.VMEM((160, CH), jnp.float32),
            pltpu.VMEM((160, CH), jnp.float32),
            pltpu.VMEM((160, CH), jnp.float32),
            pltpu.SemaphoreType.DMA,
            pltpu.SemaphoreType.DMA,
            pltpu.SemaphoreType.DMA,
            pltpu.SemaphoreType.DMA,
        ],
    )
    def body(h1s_r, dsrow_r, src_r, dst_r, dexp_r, z_r, t2_r,
             sh_u, sh_acc, v_src, v_dst, v_dsrow, v_dexp, v_rows, v_rows2,
             v_w, v_txA, v_txB, sg0, sg1, ss0, ss1):
        c = lax.axis_index("c")
        s = lax.axis_index("s")
        co = c * CH
        r0 = s * 160

        pltpu.sync_copy(src_r.at[s], v_src)
        pltpu.sync_copy(dst_r.at[s], v_dst)
        pltpu.sync_copy(dsrow_r.at[s], v_dsrow)
        pltpu.sync_copy(dexp_r.at[pl.ds(r0, 160), :], v_dexp)
        pltpu.sync_copy(z_r, sh_acc.at[pl.ds(r0, 160), :])
        plsc.subcore_barrier()

        for i in range(5):
            pltpu.sync_copy(
                h1s_r.at[c, pl.ds(s * 640 + i * 128, 128), :], v_rows)
            pltpu.sync_copy(v_rows, sh_acc.at[v_dsrow.at[i]], add=True)
        plsc.subcore_barrier()

        def rowpass(k, w_ref, txA, txB, first):
            scale = -1.0 if k == 1 else -2.0

            def rbody(r, _):
                for m in range(CH // 16):
                    sl = pl.ds(m * 16, 16)
                    w = w_ref[r, sl]
                    dv = v_dexp[r, sl]
                    tx = scale * dv * w
                    if not first and k > 1:
                        tx = tx - txA[r, sl]
                    txA[r, sl] = tx
                    w_ref[r, sl] = dv * tx
                return 0
            lax.fori_loop(0, 160, rbody, 0)
            pltpu.sync_copy(txA, t2_r.at[k, c, pl.ds(r0, 160), :])
            pltpu.sync_copy(w_ref, sh_u.at[pl.ds(r0, 160), :])

        pltpu.sync_copy(sh_acc.at[pl.ds(r0, 160), :], v_txB)
        pltpu.sync_copy(v_txB, t2_r.at[0, c, pl.ds(r0, 160), :])

        def r0body(r, _):
            for m in range(CH // 16):
                sl = pl.ds(m * 16, 16)
                v_w[r, sl] = v_dexp[r, sl] * v_txB[r, sl]
            return 0
        lax.fori_loop(0, 160, r0body, 0)
        pltpu.sync_copy(v_w, sh_u.at[pl.ds(r0, 160), :])
        plsc.subcore_barrier()

        txA, txB = v_txA, v_txB
        for k in range(1, K):
            pltpu.sync_copy(z_r, sh_acc.at[pl.ds(r0, 160), :])
            plsc.subcore_barrier()

            pltpu.async_copy(sh_u.at[v_src.at[0]], v_rows, sg0)
            pltpu.async_copy(sh_u.at[v_src.at[1]], v_rows2, sg1)

            def spmm_pair(i, _):
                a = 2 * i
                pltpu.make_async_copy(sh_u.at[v_src.at[a]], v_rows, sg0).wait()
                pltpu.async_copy(v_rows, sh_acc.at[v_dst.at[a]], ss0, add=True)
                pltpu.make_async_copy(
                    sh_u.at[v_src.at[a + 1]], v_rows2, sg1).wait()
                pltpu.async_copy(
                    v_rows2, sh_acc.at[v_dst.at[a + 1]], ss1, add=True)

                @pl.when(i < 19)
                def _nxt():
                    pltpu.make_async_copy(
                        v_rows, sh_acc.at[v_dst.at[a]], ss0).wait()
                    pltpu.async_copy(sh_u.at[v_src.at[a + 2]], v_rows, sg0)
                    pltpu.make_async_copy(
                        v_rows2, sh_acc.at[v_dst.at[a + 1]], ss1).wait()
                    pltpu.async_copy(sh_u.at[v_src.at[a + 3]], v_rows2, sg1)

                @pl.when(i == 19)
                def _fin():
                    pltpu.make_async_copy(
                        v_rows, sh_acc.at[v_dst.at[a]], ss0).wait()
                    pltpu.make_async_copy(
                        v_rows2, sh_acc.at[v_dst.at[a + 1]], ss1).wait()
                return 0
            lax.fori_loop(0, 20, spmm_pair, 0)
            plsc.subcore_barrier()

            pltpu.sync_copy(sh_acc.at[pl.ds(r0, 160), :], v_w)
            rowpass(k, v_w, txA, txB, first=(k == 1))
            txA, txB = txB, txA
            plsc.subcore_barrier()

    return body(h1s, dsrow2d, src2d, dst2d, dinv2exp, zrows)


def _tc_head(T2, W2, b2, Wh, bh, Wlp, blp):
    def body(t2_ref, w2_ref, b2_ref, wh_ref, bh_ref, wl_ref, bl_ref, o_ref):
        acc = jnp.zeros((N2P, 128), jnp.float32)
        for k in range(K):
            w2k = w2_ref[k]
            for c in range(2):
                acc = acc + jax.lax.dot_general(
                    t2_ref[k, c], w2k[c * CH:(c + 1) * CH, :],
                    (((1,), (0,)), ((), ())),
                    preferred_element_type=jnp.float32)
        out2 = acc + b2_ref[...]
        Hm = jax.lax.dot_general(out2, wh_ref[...], (((1,), (0,)), ((), ())),
                                 preferred_element_type=jnp.float32)
        Hm = Hm + bh_ref[...]
        rows = jax.lax.broadcasted_iota(jnp.int32, (N2P, H), 0)
        Hm = jnp.where(rows < N2, Hm, -3.0e38)
        m = jnp.max(Hm, axis=0, keepdims=True)
        Z = jax.lax.dot_general(m, wl_ref[...], (((1,), (0,)), ((), ())),
                                preferred_element_type=jnp.float32)
        o_ref[...] = Z + bl_ref[...]

    return pl.pallas_call(
        body,
        out_shape=jax.ShapeDtypeStruct((1, 128), jnp.float32),
    )(T2, W2, b2.reshape(1, 128), Wh, bh.reshape(1, H), Wlp, blp)


E1P = 327680


def _rsqrt16(x):
    i = plsc.bitcast(x, jnp.int32)
    i = jnp.int32(0x5F3759DF) - jax.lax.shift_right_logical(i, 1)
    y = plsc.bitcast(i, jnp.float32)
    for _ in range(4):
        y = y * (1.5 - 0.5 * x * y * y)
    return jnp.where(x > 0.0, y, 0.0)


def _sc_conv1_prep(xT, src1, dst1, src2, z80, id80, id20):
    mesh = plsc.VectorSubcoreMesh(core_axis_name="c", subcore_axis_name="s")

    @functools.partial(
        pl.kernel,
        mesh=mesh,
        out_type=(jax.ShapeDtypeStruct((K, 3, 80, 128), jnp.float32),
                  jax.ShapeDtypeStruct((N2P, CH), jnp.float32)),
        compiler_params=pltpu.CompilerParams(use_tc_tiling_on_sc=False,
                                             needs_layout_passes=False),
        scratch_types=(
            [pltpu.VMEM_SHARED((80, 128), jnp.float32)] * 6 +
            [pltpu.VMEM((80, 128), jnp.float32)] * 6 +
            [pltpu.VMEM((160, 128), jnp.int32)] * 2 +
            [pltpu.VMEM((1, 80), jnp.int32),
             pltpu.VMEM((1, 20), jnp.int32),
             pltpu.VMEM((5, 128), jnp.float32),
             pltpu.VMEM((5, 128), jnp.float32),
             pltpu.VMEM((5, 128), jnp.float32)] +
            [pltpu.VMEM((5, 128), jnp.float32)] * 6 +
            [pltpu.VMEM((128, CH), jnp.float32)]
        ),
    )
    def body(xT_r, src_r, dst_r, src2_r, z_r, id80_r, id20_r, t1_r, dexp_r,
             shu0, shu1, shu2, sha0, sha1, sha2,
             vx0, vx1, vx2, va0, va1, va2,
             v_src, v_dst, v_id80, v_id20, v_w, v_dinv, v_u,
             txA0, txA1, txA2, txB0, txB1, txB2, v_dexp):
        cidx = lax.axis_index("c")
        s = lax.axis_index("s")
        shu = (shu0, shu1, shu2)
        sha = (sha0, sha1, sha2)
        vx = (vx0, vx1, vx2)
        va = (va0, va1, va2)
        sl5 = pl.ds(5 * s, 5)

        def groups(i, fn):
            for j in range(8):
                g = pl.ds(16 * j, 16)
                s16 = v_src[i, g]
                d16 = v_dst[i, g]
                shi = jax.lax.shift_right_logical(s16, 7)
                slo = jax.lax.bitwise_and(s16, 127)
                dhi = jax.lax.shift_right_logical(d16, 7)
                dlo = jax.lax.bitwise_and(d16, 127)
                fn(shi, slo, dhi, dlo)

        ones16 = jnp.full((16,), 1.0, jnp.float32)

        def conv1_program(chans):
            nch = len(chans)
            pltpu.sync_copy(src_r.at[s], v_src)
            pltpu.sync_copy(dst_r.at[s], v_dst)
            pltpu.sync_copy(id80_r, v_id80)
            pltpu.sync_copy(z_r, va0)

            def histbody(i, _):
                groups(i, lambda shi, slo, dhi, dlo:
                       plsc.addupdate_scatter(va0, [shi, slo], ones16))
                return 0
            lax.fori_loop(0, 160, histbody, 0)
            pltpu.sync_copy(z_r.at[pl.ds(0, 5)], sha0.at[sl5])
            plsc.subcore_barrier()
            pltpu.sync_copy(va0, sha0.at[v_id80.at[0]], add=True)
            plsc.subcore_barrier()
            pltpu.sync_copy(sha0.at[sl5], v_w)

            def newton(r, _):
                for m in range(8):
                    g = pl.ds(16 * m, 16)
                    v_dinv[r, g] = _rsqrt16(v_w[r, g])
                return 0
            lax.fori_loop(0, 5, newton, 0)
            txA = [txA0, txA1, txA2][:nch]
            txB = [txB0, txB1, txB2][:nch]
            for j, ch in enumerate(chans):
                pltpu.sync_copy(xT_r.at[ch, sl5], txB[j])
                pltpu.sync_copy(txB[j], t1_r.at[0, ch, sl5])

                def u0body(r, _, _b=txB[j]):
                    for m in range(8):
                        g = pl.ds(16 * m, 16)
                        v_u[r, g] = v_dinv[r, g] * _b[r, g]
                    return 0
                lax.fori_loop(0, 5, u0body, 0)
                pltpu.sync_copy(v_u, shu[j].at[sl5])
            plsc.subcore_barrier()

            for k in range(1, K):
                for j in range(nch):
                    pltpu.sync_copy(z_r, va[j])
                    pltpu.sync_copy(z_r.at[pl.ds(0, 5)], sha[j].at[sl5])
                    pltpu.sync_copy(shu[j], vx[j])
                plsc.subcore_barrier()

                def edgebody(i, _):
                    def work(shi, slo, dhi, dlo):
                        for j in range(nch):
                            g = plsc.load_gather(vx[j], [shi, slo])
                            plsc.addupdate_scatter(va[j], [dhi, dlo], g)
                    groups(i, work)
                    return 0
                lax.fori_loop(0, 160, edgebody, 0)
                for j in range(nch):
                    pltpu.sync_copy(va[j], sha[j].at[v_id80.at[0]], add=True)
                plsc.subcore_barrier()

                scale = -1.0 if k == 1 else -2.0
                for j, ch in enumerate(chans):
                    pltpu.sync_copy(sha[j].at[sl5], v_w)

                    def rowbody(r, _, _a=txA[j], _k=k):
                        for m in range(8):
                            g = pl.ds(16 * m, 16)
                            tx = scale * v_dinv[r, g] * v_w[r, g]
                            if _k > 1:
                                tx = tx - _a[r, g]
                            _a[r, g] = tx
                            v_u[r, g] = v_dinv[r, g] * tx
                        return 0
                    lax.fori_loop(0, 5, rowbody, 0)
                    pltpu.sync_copy(txA[j], t1_r.at[k, ch, sl5])
                    pltpu.sync_copy(v_u, shu[j].at[sl5])
                txA, txB = txB, txA
                plsc.subcore_barrier()

        @pl.when(cidx == 0)
        def _core0():
            conv1_program([0, 1])

        @pl.when(cidx == 1)
        def _core1():
            pltpu.sync_copy(src2_r.at[s], v_src.at[pl.ds(0, 40)])
            pltpu.sync_copy(id20_r, v_id20)
            pltpu.sync_copy(z_r, va0)

            def histbody(i, _):
                groups(i, lambda shi, slo, dhi, dlo:
                       plsc.addupdate_scatter(va0, [shi, slo], ones16))
                return 0
            lax.fori_loop(0, 40, histbody, 0)

            @pl.when(s < 10)
            def _z():
                pltpu.sync_copy(z_r.at[pl.ds(0, 2)], sha0.at[pl.ds(2 * s, 2)])
            plsc.subcore_barrier()
            pltpu.sync_copy(va0.at[pl.ds(0, 20)], sha0.at[v_id20.at[0]], add=True)
            plsc.subcore_barrier()

            @pl.when(s < 10)
            def _dexp():
                pltpu.sync_copy(sha0.at[pl.ds(2 * s, 2)], v_w.at[pl.ds(0, 2)])
                for r in range(2):
                    for m in range(8):
                        g = pl.ds(16 * m, 16)
                        v_dinv[r, g] = _rsqrt16(v_w[r, g])
                for half in range(2):
                    def expand(j, _):
                        sp = plsc.load_gather(
                            v_dinv, [jnp.full((16,), half, jnp.int32),
                                     jnp.full((16,), j, jnp.int32)])
                        for m in range(CH // 16):
                            v_dexp[j, pl.ds(16 * m, 16)] = sp
                        return 0
                    lax.fori_loop(0, 128, expand, 0)
                    pltpu.sync_copy(
                        v_dexp, dexp_r.at[pl.ds(256 * s + 128 * half, 128), :])
            conv1_program([2])

    return body(xT, src1, dst1, src2, z80, id80, id20)


def _tc_mid(T1r18, W1s, b1, dsvp):
    def bodyfn(x_ref, w_ref, b_ref, d_ref, o_ref):
        h = jax.lax.dot_general(x_ref[...], w_ref[...],
                                (((0,), (0,)), ((), ())),
                                preferred_element_type=jnp.float32)
        o_ref[...] = jnp.maximum(h + b_ref[...], 0.0) * d_ref[...]

    return pl.pallas_call(
        bodyfn,
        out_shape=jax.ShapeDtypeStruct((N1P, 128), jnp.float32),
    )(T1r18, W1s, b1.reshape(1, 128), dsvp)


def kernel(pos, edge_index0, edge_index1, ds_row, ds_col, ds_values,
           W1, b1, W2, b2, Wh, bh, Wl, bl):
    i32 = jnp.int32

    xT = jnp.zeros((3, N1P), jnp.float32).at[:, :N1].set(
        pos.T).reshape(3, 80, 128)
    src1 = jnp.full((E1P,), N1, i32).at[:E1].set(
        edge_index0[0].astype(i32)).reshape(NT, 160, 128)
    dst1 = jnp.full((E1P,), N1, i32).at[:E1].set(
        edge_index0[1].astype(i32)).reshape(NT, 160, 128)
    dsrow2d = jnp.full((N1P,), N2, i32).at[:N1].set(
        ds_row.astype(i32)).reshape(NT, 5, 128)
    src2 = jnp.full((E2P,), N2, i32).at[:E2].set(
        edge_index1[0].astype(i32)).reshape(NT, 40, 128)
    dst2 = jnp.full((E2P,), N2, i32).at[:E2].set(
        edge_index1[1].astype(i32)).reshape(NT, 40, 128)
    z80 = jnp.zeros((80, 128), jnp.float32)
    id80 = jnp.arange(80, dtype=i32).reshape(1, 80)
    id20 = jnp.arange(20, dtype=i32).reshape(1, 20)
    zrows = jnp.zeros((160, CH), jnp.float32)
    dsvp = jnp.zeros((N1P, 1), jnp.float32).at[:N1, 0].set(ds_values)

    T1, dinv2exp = _sc_conv1_prep(xT, src1, dst1, src2, z80, id80, id20)

    W1s = W1.reshape(K * 3, 128)
    h1 = _tc_mid(T1.reshape(K * 3, N1P), W1s, b1, dsvp)
    h1s = jnp.stack([h1[:, :CH], h1[:, CH:]])

    T2 = _sc_pool_conv2(h1s, dsrow2d, src2, dst2, dinv2exp, zrows)

    Wlp = jnp.zeros((H, 128), jnp.float32).at[:, :NUM_CLASSES].set(Wl)
    blp = jnp.zeros((1, 128), jnp.float32).at[0, :NUM_CLASSES].set(bl)
    Zp = _tc_head(T2, W2, b2, Wh, bh, Wlp, blp)
    return Zp[0, :NUM_CLASSES]
